# Initial kernel scaffold; baseline (speedup 1.0000x reference)
import jax
import jax.numpy as jnp
from jax.experimental import pallas as pl


def kernel(fwd_x, fwd_edge_index, fwd_edge_attr, bwd_x, bwd_edge_index, bwd_edge_attr, embed, W_ih_f, W_hh_f, b_ih_f, b_hh_f, W_ih_r, W_hh_r, b_ih_r, b_hh_r, lin1_W, lin1_b, lin2_W, lin2_b, fc_Wl, fc_bl, fc_Wr, fc_br, fc_att, fc_bias, bc_Wl, bc_bl, bc_Wr, bc_br, bc_att, bc_bias):
    raise NotImplementedError("write your pallas kernel here")


# speedup vs baseline: 19.5221x; 19.5221x over previous
import jax
import jax.numpy as jnp
from jax.experimental import pallas as pl

STATE_ID = 53
IN_IDX = 161
OUT_IDX = 320
HID = 373
NP = 10240
CP = 384
SEQ = 20
NE = 160000
E_BLK = 2048
E_TOT = 327680
RB = 1024


def _lstm_body(tok_ref, embed_ref, wih_f_t_ref, whh_f_t_ref, b_f_ref,
               wih_r_t_ref, b_r_ref, lin1_t_ref, l1b_ref, lin2_ref,
               l2b_ref, out_ref):
    B = tok_ref.shape[0]
    H = 32
    f32 = jnp.float32
    bf16 = jnp.bfloat16
    ew_f = jnp.dot(embed_ref[...], wih_f_t_ref[...],
                   preferred_element_type=f32).astype(bf16)
    ew_r = jnp.dot(embed_ref[...], wih_r_t_ref[...],
                   preferred_element_type=f32).astype(bf16)
    whh_t = whh_f_t_ref[...].astype(bf16)
    iota_v = jax.lax.broadcasted_iota(jnp.int32, (B, 128), 1)
    tok = tok_ref[...]

    def onehot(t):
        return (tok[:, t][:, None] == iota_v).astype(bf16)

    def gates_act(g, c_prev):
        ii = g[:, 0:H]
        ff = g[:, H:2 * H]
        gg = g[:, 2 * H:3 * H]
        oo = g[:, 3 * H:4 * H]
        c = jax.nn.sigmoid(ff) * c_prev + jax.nn.sigmoid(ii) * jnp.tanh(gg)
        h = jax.nn.sigmoid(oo) * jnp.tanh(c)
        return h, c

    b_f = b_f_ref[...]
    h = jnp.zeros((B, H), f32)
    c = jnp.zeros((B, H), f32)
    for t in range(SEQ):
        g = jnp.dot(onehot(t), ew_f, preferred_element_type=f32)
        g = g + jnp.dot(h.astype(bf16), whh_t, preferred_element_type=f32)
        g = g + b_f
        h, c = gates_act(g, c)
    g_r = jnp.dot(onehot(SEQ - 1), ew_r, preferred_element_type=f32)
    g_r = g_r + b_r_ref[...]
    h_r, _ = gates_act(g_r, jnp.zeros((B, H), f32))

    feat = jnp.concatenate([h, h_r], axis=1).astype(bf16)
    e1 = jnp.maximum(
        jnp.dot(feat, lin1_t_ref[...].astype(bf16),
                preferred_element_type=f32) + l1b_ref[...], 0.0)
    es = jnp.sum(e1 * lin2_ref[...], axis=1) + l2b_ref[0, 0]
    out_ref[...] = jnp.maximum(es, 0.0)


def _lstm_escalar(tokens, embed, wih_f_t, whh_f_t, b_f, wih_r_t, b_r,
                  lin1_t, lin1_b, lin2_W, lin2_b):
    nblk = E_TOT // E_BLK

    def full(*shape):
        return pl.BlockSpec(shape, lambda i: (0,) * len(shape))

    return pl.pallas_call(
        _lstm_body,
        grid=(nblk,),
        in_specs=[
            pl.BlockSpec((E_BLK, SEQ), lambda i: (i, 0)),
            full(128, 32),
            full(32, 128), full(32, 128), full(1, 128),
            full(32, 128), full(1, 128),
            full(64, 32), full(1, 32),
            full(1, 32), full(1, 1),
        ],
        out_specs=pl.BlockSpec((E_BLK,), lambda i: (i,)),
        out_shape=jax.ShapeDtypeStruct((E_TOT,), jnp.float32),
    )(tokens, embed, wih_f_t, whh_f_t, b_f, wih_r_t, b_r, lin1_t, lin1_b,
      lin2_W, lin2_b)


def _argmax_body(x_ref, out_ref):
    v = x_ref[0]
    iota_c = jax.lax.broadcasted_iota(jnp.int32, v.shape, 1)
    v = jnp.where(iota_c < STATE_ID, v, -1e30)
    m = jnp.max(v, axis=1, keepdims=True)
    idx = jnp.min(jnp.where(v >= m, iota_c, 128), axis=1)
    out_ref[...] = idx.reshape(1, 1, NP)


def _node_argmax(x2_head):
    out = pl.pallas_call(
        _argmax_body,
        grid=(2,),
        in_specs=[pl.BlockSpec((1, NP, 128), lambda b: (b, 0, 0))],
        out_specs=pl.BlockSpec((1, 1, NP), lambda b: (b, 0, 0)),
        out_shape=jax.ShapeDtypeStruct((2, 1, NP), jnp.int32),
    )(x2_head)
    return out.reshape(2, NP)


def _proj_body(x_ref, w_ref, b_ref, lo_ref, hi_ref):
    y = jax.lax.dot_general(x_ref[...], w_ref[0, 0],
                            (((1,), (1,)), ((), ())),
                            preferred_element_type=jnp.float32)
    y = y + b_ref[0, 0]
    lo_ref[...] = y[:, :192]
    hi_ref[...] = y[:, 192:]


def _projections(x_upd, W_all, b_all):
    nr = NP // RB
    return pl.pallas_call(
        _proj_body,
        grid=(2, 2, nr),
        in_specs=[
            pl.BlockSpec((RB, CP), lambda b, s, r: (b * nr + r, 0)),
            pl.BlockSpec((1, 1, CP, CP), lambda b, s, r: (b, s, 0, 0)),
            pl.BlockSpec((1, 1, 1, CP), lambda b, s, r: (b, s, 0, 0)),
        ],
        out_specs=[
            pl.BlockSpec((RB, 192), lambda b, s, r: ((b * 2 + s) * nr + r, 0)),
            pl.BlockSpec((RB, 192), lambda b, s, r: ((b * 2 + s) * nr + r, 0)),
        ],
        out_shape=[jax.ShapeDtypeStruct((4 * NP, 192), jnp.float32)] * 2,
    )(x_upd, W_all, b_all)


def _final_body(lo_ref, hi_ref, den_ref, x_ref, bias_ref, out_ref):
    acc = jnp.concatenate([lo_ref[...], hi_ref[...]], axis=1)
    den = den_ref[...].reshape(RB, 1)
    gat = jnp.where(den > 0.0, acc / den, 0.0) + bias_ref[0, 0]
    out_ref[...] = jnp.maximum(gat + x_ref[...], 0.0)


def _final_combine(acc_lo, acc_hi, denom, x_upd, bias2):
    nr = NP // RB
    return pl.pallas_call(
        _final_body,
        grid=(2, nr),
        in_specs=[
            pl.BlockSpec((RB, 192), lambda b, r: (b * nr + r, 0)),
            pl.BlockSpec((RB, 192), lambda b, r: (b * nr + r, 0)),
            pl.BlockSpec((RB,), lambda b, r: (b * nr + r,)),
            pl.BlockSpec((RB, CP), lambda b, r: (b * nr + r, 0)),
            pl.BlockSpec((1, 1, CP), lambda b, r: (b, 0, 0)),
        ],
        out_specs=pl.BlockSpec((RB, CP), lambda b, r: (b * nr + r, 0)),
        out_shape=jax.ShapeDtypeStruct((2 * NP, CP), jnp.float32),
    )(acc_lo, acc_hi, denom, x_upd, bias2)


def _pad2(a):
    return jnp.pad(a, ((0, CP - HID), (0, CP - HID)))


def _pad1(a):
    return jnp.pad(a, (0, CP - HID))


def kernel(fwd_x, fwd_edge_index, fwd_edge_attr, bwd_x, bwd_edge_index,
           bwd_edge_attr, embed, W_ih_f, W_hh_f, b_ih_f, b_hh_f, W_ih_r,
           W_hh_r, b_ih_r, b_hh_r, lin1_W, lin1_b, lin2_W, lin2_b, fc_Wl,
           fc_bl, fc_Wr, fc_br, fc_att, fc_bias, bc_Wl, bc_bl, bc_Wr, bc_br,
           bc_att, bc_bias):
    f32 = jnp.float32
    n = fwd_x.shape[0]

    tokens = jnp.concatenate([fwd_edge_attr, bwd_edge_attr], axis=0)
    tokens = jnp.pad(tokens, ((0, E_TOT - 2 * NE), (0, 0))).astype(jnp.int32)
    e_scalar = _lstm_escalar(
        tokens, embed, W_ih_f.T, W_hh_f.T, (b_ih_f + b_hh_f).reshape(1, -1),
        W_ih_r.T, (b_ih_r + b_hh_r).reshape(1, -1), lin1_W.T,
        lin1_b.reshape(1, -1), lin2_W, lin2_b.reshape(1, 1))

    x2 = jnp.stack([jnp.pad(fwd_x, ((0, NP - n), (0, CP - HID))),
                    jnp.pad(bwd_x, ((0, NP - n), (0, CP - HID)))])
    nid = _node_argmax(x2[:, :, :128])

    src = jnp.stack([fwd_edge_index[0], bwd_edge_index[0]])
    dst = jnp.stack([fwd_edge_index[1], bwd_edge_index[1]])

    es2 = e_scalar[:2 * NE].reshape(2, NE)
    xs = []
    for b in range(2):
        sid = nid[b][src[b]]
        tid = nid[b][dst[b]]
        xb = x2[b]
        xb = xb.at[dst[b], IN_IDX + sid].set(es2[b])
        xb = xb.at[src[b], OUT_IDX + tid].set(es2[b])
        xs.append(xb)
    x_upd = jnp.concatenate(xs, axis=0)

    W_all = jnp.stack([jnp.stack([_pad2(fc_Wl), _pad2(fc_Wr)]),
                       jnp.stack([_pad2(bc_Wl), _pad2(bc_Wr)])])
    b_all = jnp.stack([jnp.stack([_pad1(fc_bl), _pad1(fc_br)]),
                       jnp.stack([_pad1(bc_bl), _pad1(bc_br)])])
    lo, hi = _projections(x_upd, W_all, b_all.reshape(2, 2, 1, CP))

    att2 = jnp.stack([_pad1(fc_att), _pad1(bc_att)])

    acc_parts, den_parts = [], []
    for b in range(2):
        xl = jnp.concatenate([lo[(2 * b) * NP:(2 * b + 1) * NP],
                              hi[(2 * b) * NP:(2 * b + 1) * NP]], axis=1)
        xr = jnp.concatenate([lo[(2 * b + 1) * NP:(2 * b + 2) * NP],
                              hi[(2 * b + 1) * NP:(2 * b + 2) * NP]], axis=1)
        m = jax.nn.leaky_relu(xl[src[b]] + xr[dst[b]], negative_slope=0.2)
        e = m @ att2[b]
        ex = jnp.exp(e)
        den_parts.append(jax.ops.segment_sum(ex, dst[b], num_segments=NP))
        w = ex[:, None] * xl[src[b]]
        acc_parts.append(jax.ops.segment_sum(w, dst[b], num_segments=NP))
    acc = jnp.concatenate(acc_parts, axis=0)
    denom = jnp.concatenate(den_parts, axis=0)

    bias2 = jnp.stack([_pad1(fc_bias), _pad1(bc_bias)]).reshape(2, 1, CP)
    y = _final_combine(acc[:, :192], acc[:, 192:], denom, x_upd, bias2)
    return jnp.concatenate([y[:n, :HID], y[NP:NP + n, :HID]], axis=1)


# speedup vs baseline: 29.4115x; 1.5066x over previous
import functools

import jax
import jax.numpy as jnp
from jax import lax
from jax.experimental import pallas as pl
from jax.experimental.pallas import tpu as pltpu
from jax.experimental.pallas import tpu_sc as plsc

STATE_ID = 53
IN_IDX = 161
OUT_IDX = 320
HID = 373
NP = 10240
CP = 384
SEQ = 20
NE = 160000
NEP = 163840
E_TOT = 2 * NEP
E_BLK = 2048
RB = 1024
NROW = E_TOT // 128


def _lstm_body(tok_ref, embed_ref, wih_f_t_ref, whh_f_t_ref, b_f_ref,
               wih_r_t_ref, b_r_ref, lin1_t_ref, l1b_ref, lin2_ref,
               l2b_ref, out_ref):
    B = tok_ref.shape[0]
    H = 32
    f32 = jnp.float32
    bf16 = jnp.bfloat16
    ew_f = jnp.dot(embed_ref[...], wih_f_t_ref[...],
                   preferred_element_type=f32).astype(bf16)
    ew_r = jnp.dot(embed_ref[...], wih_r_t_ref[...],
                   preferred_element_type=f32).astype(bf16)
    whh_t = whh_f_t_ref[...].astype(bf16)
    iota_v = jax.lax.broadcasted_iota(jnp.int32, (B, 128), 1)
    tok = tok_ref[...]

    def onehot(t):
        return (tok[:, t][:, None] == iota_v).astype(bf16)

    def gates_act(g, c_prev):
        ii = g[:, 0:H]
        ff = g[:, H:2 * H]
        gg = g[:, 2 * H:3 * H]
        oo = g[:, 3 * H:4 * H]
        c = jax.nn.sigmoid(ff) * c_prev + jax.nn.sigmoid(ii) * jnp.tanh(gg)
        h = jax.nn.sigmoid(oo) * jnp.tanh(c)
        return h, c

    b_f = b_f_ref[...]
    h = jnp.zeros((B, H), f32)
    c = jnp.zeros((B, H), f32)
    for t in range(SEQ):
        g = jnp.dot(onehot(t), ew_f, preferred_element_type=f32)
        g = g + jnp.dot(h.astype(bf16), whh_t, preferred_element_type=f32)
        g = g + b_f
        h, c = gates_act(g, c)
    g_r = jnp.dot(onehot(SEQ - 1), ew_r, preferred_element_type=f32)
    g_r = g_r + b_r_ref[...]
    h_r, _ = gates_act(g_r, jnp.zeros((B, H), f32))

    feat = jnp.concatenate([h, h_r], axis=1).astype(bf16)
    e1 = jnp.maximum(
        jnp.dot(feat, lin1_t_ref[...].astype(bf16),
                preferred_element_type=f32) + l1b_ref[...], 0.0)
    es = jnp.sum(e1 * lin2_ref[...], axis=1) + l2b_ref[0, 0]
    out_ref[...] = jnp.maximum(es, 0.0)


def _lstm_escalar(tokens, embed, wih_f_t, whh_f_t, b_f, wih_r_t, b_r,
                  lin1_t, lin1_b, lin2_W, lin2_b):
    nblk = E_TOT // E_BLK

    def full(*shape):
        return pl.BlockSpec(shape, lambda i: (0,) * len(shape))

    return pl.pallas_call(
        _lstm_body,
        grid=(nblk,),
        in_specs=[
            pl.BlockSpec((E_BLK, SEQ), lambda i: (i, 0)),
            full(128, 32),
            full(32, 128), full(32, 128), full(1, 128),
            full(32, 128), full(1, 128),
            full(64, 32), full(1, 32),
            full(1, 32), full(1, 1),
        ],
        out_specs=pl.BlockSpec((E_BLK,), lambda i: (i,)),
        out_shape=jax.ShapeDtypeStruct((E_TOT,), jnp.float32),
    )(tokens, embed, wih_f_t, whh_f_t, b_f, wih_r_t, b_r, lin1_t, lin1_b,
      lin2_W, lin2_b)


def _argmax_body(x_ref, out_ref):
    v = x_ref[0]
    iota_c = jax.lax.broadcasted_iota(jnp.int32, v.shape, 1)
    v = jnp.where(iota_c < STATE_ID, v, -1e30)
    m = jnp.max(v, axis=1, keepdims=True)
    idx = jnp.min(jnp.where(v >= m, iota_c, 128), axis=1)
    out_ref[...] = idx.reshape(1, 1, NP)


def _node_argmax(x2_head):
    out = pl.pallas_call(
        _argmax_body,
        grid=(2,),
        in_specs=[pl.BlockSpec((1, NP, 128), lambda b: (b, 0, 0))],
        out_specs=pl.BlockSpec((1, 1, NP), lambda b: (b, 0, 0)),
        out_shape=jax.ShapeDtypeStruct((2, 1, NP), jnp.int32),
    )(x2_head)
    return out.reshape(2, NP)


def _proj_body(x_ref, w_ref, b_ref, lo_ref, hi_ref):
    y = jax.lax.dot_general(x_ref[...], w_ref[0, 0],
                            (((1,), (1,)), ((), ())),
                            preferred_element_type=jnp.float32)
    y = y + b_ref[0, 0]
    lo_ref[...] = y[:, :192]
    hi_ref[...] = y[:, 192:]


def _projections(x_upd, W_all, b_all):
    nr = NP // RB
    return pl.pallas_call(
        _proj_body,
        grid=(2, 2, nr),
        in_specs=[
            pl.BlockSpec((RB, CP), lambda b, s, r: (b * nr + r, 0)),
            pl.BlockSpec((1, 1, CP, CP), lambda b, s, r: (b, s, 0, 0)),
            pl.BlockSpec((1, 1, 1, CP), lambda b, s, r: (b, s, 0, 0)),
        ],
        out_specs=[
            pl.BlockSpec((RB, 192), lambda b, s, r: ((b * 2 + s) * nr + r, 0)),
            pl.BlockSpec((RB, 192), lambda b, s, r: ((b * 2 + s) * nr + r, 0)),
        ],
        out_shape=[jax.ShapeDtypeStruct((4 * NP, 192), jnp.float32)] * 2,
    )(x_upd, W_all, b_all)


def _final_body(lo_ref, hi_ref, den_ref, x_ref, bias_ref, out_ref):
    acc = jnp.concatenate([lo_ref[...], hi_ref[...]], axis=1)
    den = den_ref[...].reshape(RB, 1)
    gat = jnp.where(den > 0.0, acc / den, 0.0) + bias_ref[0, 0]
    out_ref[...] = jnp.maximum(gat + x_ref[...], 0.0)


def _final_combine(acc_lo, acc_hi, denom, x_upd, bias2):
    nr = NP // RB
    return pl.pallas_call(
        _final_body,
        grid=(2, nr),
        in_specs=[
            pl.BlockSpec((RB, 192), lambda b, r: (b * nr + r, 0)),
            pl.BlockSpec((RB, 192), lambda b, r: (b * nr + r, 0)),
            pl.BlockSpec((RB,), lambda b, r: (b * nr + r,)),
            pl.BlockSpec((RB, CP), lambda b, r: (b * nr + r, 0)),
            pl.BlockSpec((1, 1, CP), lambda b, r: (b, 0, 0)),
        ],
        out_specs=pl.BlockSpec((RB, CP), lambda b, r: (b * nr + r, 0)),
        out_shape=jax.ShapeDtypeStruct((2 * NP, CP), jnp.float32),
    )(acc_lo, acc_hi, denom, x_upd, bias2)


def _sc_scatter_body(x_hbm, src_hbm, dst_hbm, es_hbm, nid_hbm, out_hbm,
                     nid_v, srcv, dstv, esv, pos1v, pos2v, cbuf, sem):
    c = lax.axis_index("c")
    t = lax.axis_index("s")
    half = NP * CP
    per_tile = half // 16
    ch = 49152
    base = c * half + t * per_tile

    def copy_chunk(i, _):
        pltpu.sync_copy(x_hbm.at[pl.ds(base + i * ch, ch)], cbuf)
        pltpu.sync_copy(cbuf, out_hbm.at[pl.ds(base + i * ch, ch)])
        return 0

    lax.fori_loop(0, per_tile // ch, copy_chunk, 0)
    plsc.subcore_barrier()

    pltpu.sync_copy(nid_hbm.at[pl.ds(c * NP, NP)], nid_v)
    row0 = c * (NROW // 2) + t * 80
    bflat = c * half
    for ci in range(5):
        r = row0 + ci * 16
        pltpu.sync_copy(src_hbm.at[pl.ds(r, 16)], srcv)
        pltpu.sync_copy(dst_hbm.at[pl.ds(r, 16)], dstv)
        pltpu.sync_copy(es_hbm.at[pl.ds(r, 16)], esv)

        def compute(tt, _):
            j = tt // 8
            sl = pl.ds((tt % 8) * 16, 16)
            s16 = srcv[j, sl]
            d16 = dstv[j, sl]
            sid = plsc.load_gather(nid_v, [s16])
            tid = plsc.load_gather(nid_v, [d16])
            pos1v[j, sl] = bflat + d16 * CP + (IN_IDX + sid)
            pos2v[j, sl] = bflat + s16 * CP + (OUT_IDX + tid)
            return 0

        lax.fori_loop(0, 128, compute, 0)
        handles = []
        for j in range(16):
            handles.append(
                pltpu.async_copy(esv.at[j], out_hbm.at[pos1v.at[j]], sem))
            handles.append(
                pltpu.async_copy(esv.at[j], out_hbm.at[pos2v.at[j]], sem))
        for h in handles:
            h.wait()


def _sc_scatter(x_flat, src2d, dst2d, es2d, nid_flat):
    i32 = jnp.int32
    f32 = jnp.float32
    mesh = plsc.VectorSubcoreMesh(core_axis_name="c", subcore_axis_name="s")
    f = pl.kernel(
        _sc_scatter_body,
        out_type=jax.ShapeDtypeStruct((2 * NP * CP,), f32),
        mesh=mesh,
        compiler_params=pltpu.CompilerParams(needs_layout_passes=False),
        scratch_types=[
            pltpu.VMEM((NP,), i32),
            pltpu.VMEM((16, 128), i32), pltpu.VMEM((16, 128), i32),
            pltpu.VMEM((16, 128), f32),
            pltpu.VMEM((16, 128), i32), pltpu.VMEM((16, 128), i32),
            pltpu.VMEM((49152,), f32),
            pltpu.SemaphoreType.DMA,
        ],
    )
    return f(x_flat, src2d, dst2d, es2d, nid_flat)


def _pad2(a):
    return jnp.pad(a, ((0, CP - HID), (0, CP - HID)))


def _pad1(a):
    return jnp.pad(a, (0, CP - HID))


def kernel(fwd_x, fwd_edge_index, fwd_edge_attr, bwd_x, bwd_edge_index,
           bwd_edge_attr, embed, W_ih_f, W_hh_f, b_ih_f, b_hh_f, W_ih_r,
           W_hh_r, b_ih_r, b_hh_r, lin1_W, lin1_b, lin2_W, lin2_b, fc_Wl,
           fc_bl, fc_Wr, fc_br, fc_att, fc_bias, bc_Wl, bc_bl, bc_Wr, bc_br,
           bc_att, bc_bias):
    f32 = jnp.float32
    n = fwd_x.shape[0]

    tokens = jnp.concatenate(
        [jnp.pad(fwd_edge_attr, ((0, NEP - NE), (0, 0))),
         jnp.pad(bwd_edge_attr, ((0, NEP - NE), (0, 0)))],
        axis=0).astype(jnp.int32)
    e_scalar = _lstm_escalar(
        tokens, embed, W_ih_f.T, W_hh_f.T, (b_ih_f + b_hh_f).reshape(1, -1),
        W_ih_r.T, (b_ih_r + b_hh_r).reshape(1, -1), lin1_W.T,
        lin1_b.reshape(1, -1), lin2_W, lin2_b.reshape(1, 1))

    x2 = jnp.stack([jnp.pad(fwd_x, ((0, NP - n), (0, CP - HID))),
                    jnp.pad(bwd_x, ((0, NP - n), (0, CP - HID)))])
    nid = _node_argmax(x2[:, :, :128])

    src = jnp.stack([fwd_edge_index[0], bwd_edge_index[0]])
    dst = jnp.stack([fwd_edge_index[1], bwd_edge_index[1]])

    padv = (10000 + (jnp.arange(NEP - NE, dtype=jnp.int32) % 240))
    src2d = jnp.concatenate(
        [jnp.concatenate([fwd_edge_index[0].astype(jnp.int32), padv]),
         jnp.concatenate([bwd_edge_index[0].astype(jnp.int32), padv])]
    ).reshape(NROW, 128)
    dst2d = jnp.concatenate(
        [jnp.concatenate([fwd_edge_index[1].astype(jnp.int32), padv]),
         jnp.concatenate([bwd_edge_index[1].astype(jnp.int32), padv])]
    ).reshape(NROW, 128)
    es2d = e_scalar.reshape(NROW, 128)

    x_upd_flat = _sc_scatter(x2.reshape(2 * NP * CP), src2d, dst2d, es2d,
                             nid.reshape(2 * NP))
    x_upd = x_upd_flat.reshape(2 * NP, CP)

    W_all = jnp.stack([jnp.stack([_pad2(fc_Wl), _pad2(fc_Wr)]),
                       jnp.stack([_pad2(bc_Wl), _pad2(bc_Wr)])])
    b_all = jnp.stack([jnp.stack([_pad1(fc_bl), _pad1(fc_br)]),
                       jnp.stack([_pad1(bc_bl), _pad1(bc_br)])])
    lo, hi = _projections(x_upd, W_all, b_all.reshape(2, 2, 1, CP))

    att2 = jnp.stack([_pad1(fc_att), _pad1(bc_att)])

    acc_parts, den_parts = [], []
    for b in range(2):
        xl = jnp.concatenate([lo[(2 * b) * NP:(2 * b + 1) * NP],
                              hi[(2 * b) * NP:(2 * b + 1) * NP]], axis=1)
        xr = jnp.concatenate([lo[(2 * b + 1) * NP:(2 * b + 2) * NP],
                              hi[(2 * b + 1) * NP:(2 * b + 2) * NP]], axis=1)
        m = jax.nn.leaky_relu(xl[src[b]] + xr[dst[b]], negative_slope=0.2)
        e = m @ att2[b]
        ex = jnp.exp(e)
        den_parts.append(jax.ops.segment_sum(ex, dst[b], num_segments=NP))
        w = ex[:, None] * xl[src[b]]
        acc_parts.append(jax.ops.segment_sum(w, dst[b], num_segments=NP))
    acc = jnp.concatenate(acc_parts, axis=0)
    denom = jnp.concatenate(den_parts, axis=0)

    bias2 = jnp.stack([_pad1(fc_bias), _pad1(bc_bias)]).reshape(2, 1, CP)
    y = _final_combine(acc[:, :192], acc[:, 192:], denom, x_upd, bias2)
    return jnp.concatenate([y[:n, :HID], y[NP:NP + n, :HID]], axis=1)


# speedup vs baseline: 33.7049x; 1.1460x over previous
import functools

import jax
import jax.numpy as jnp
from jax import lax
from jax.experimental import pallas as pl
from jax.experimental.pallas import tpu as pltpu
from jax.experimental.pallas import tpu_sc as plsc

STATE_ID = 53
IN_IDX = 161
OUT_IDX = 320
HID = 373
NP = 10240
CP = 384
SEQ = 20
NE = 160000
NEP = 163840
E_TOT = 2 * NEP
E_BLK = 2048
RB = 1024
NROW = E_TOT // 128


def _lstm_body(tok_ref, embed_ref, wih_f_t_ref, whh_f_t_ref, b_f_ref,
               wih_r_t_ref, b_r_ref, lin1_t_ref, l1b_ref, lin2_ref,
               l2b_ref, out_ref):
    B = tok_ref.shape[0]
    H = 32
    f32 = jnp.float32
    bf16 = jnp.bfloat16
    ew_f = jnp.dot(embed_ref[...], wih_f_t_ref[...],
                   preferred_element_type=f32).astype(bf16)
    ew_r = jnp.dot(embed_ref[...], wih_r_t_ref[...],
                   preferred_element_type=f32).astype(bf16)
    whh_t = whh_f_t_ref[...].astype(bf16)
    iota_v = jax.lax.broadcasted_iota(jnp.int32, (B, 128), 1)
    tok = tok_ref[...]

    def onehot(t):
        return (tok[:, t][:, None] == iota_v).astype(bf16)

    def gates_act(g, c_prev):
        ii = g[:, 0:H]
        ff = g[:, H:2 * H]
        gg = g[:, 2 * H:3 * H]
        oo = g[:, 3 * H:4 * H]
        c = jax.nn.sigmoid(ff) * c_prev + jax.nn.sigmoid(ii) * jnp.tanh(gg)
        h = jax.nn.sigmoid(oo) * jnp.tanh(c)
        return h, c

    b_f = b_f_ref[...]
    h = jnp.zeros((B, H), f32)
    c = jnp.zeros((B, H), f32)
    for t in range(SEQ):
        g = jnp.dot(onehot(t), ew_f, preferred_element_type=f32)
        g = g + jnp.dot(h.astype(bf16), whh_t, preferred_element_type=f32)
        g = g + b_f
        h, c = gates_act(g, c)
    g_r = jnp.dot(onehot(SEQ - 1), ew_r, preferred_element_type=f32)
    g_r = g_r + b_r_ref[...]
    h_r, _ = gates_act(g_r, jnp.zeros((B, H), f32))

    feat = jnp.concatenate([h, h_r], axis=1).astype(bf16)
    e1 = jnp.maximum(
        jnp.dot(feat, lin1_t_ref[...].astype(bf16),
                preferred_element_type=f32) + l1b_ref[...], 0.0)
    es = jnp.sum(e1 * lin2_ref[...], axis=1) + l2b_ref[0, 0]
    out_ref[...] = jnp.maximum(es, 0.0)


def _lstm_escalar(tokens, embed, wih_f_t, whh_f_t, b_f, wih_r_t, b_r,
                  lin1_t, lin1_b, lin2_W, lin2_b):
    nblk = E_TOT // E_BLK

    def full(*shape):
        return pl.BlockSpec(shape, lambda i: (0,) * len(shape))

    return pl.pallas_call(
        _lstm_body,
        grid=(nblk,),
        in_specs=[
            pl.BlockSpec((E_BLK, SEQ), lambda i: (i, 0)),
            full(128, 32),
            full(32, 128), full(32, 128), full(1, 128),
            full(32, 128), full(1, 128),
            full(64, 32), full(1, 32),
            full(1, 32), full(1, 1),
        ],
        out_specs=pl.BlockSpec((E_BLK,), lambda i: (i,)),
        out_shape=jax.ShapeDtypeStruct((E_TOT,), jnp.float32),
    )(tokens, embed, wih_f_t, whh_f_t, b_f, wih_r_t, b_r, lin1_t, lin1_b,
      lin2_W, lin2_b)


def _argmax_body(x_ref, out_ref):
    v = x_ref[0]
    iota_c = jax.lax.broadcasted_iota(jnp.int32, v.shape, 1)
    v = jnp.where(iota_c < STATE_ID, v, -1e30)
    m = jnp.max(v, axis=1, keepdims=True)
    idx = jnp.min(jnp.where(v >= m, iota_c, 128), axis=1)
    out_ref[...] = idx.reshape(1, 1, NP)


def _node_argmax(x2_head):
    out = pl.pallas_call(
        _argmax_body,
        grid=(2,),
        in_specs=[pl.BlockSpec((1, NP, 128), lambda b: (b, 0, 0))],
        out_specs=pl.BlockSpec((1, 1, NP), lambda b: (b, 0, 0)),
        out_shape=jax.ShapeDtypeStruct((2, 1, NP), jnp.int32),
    )(x2_head)
    return out.reshape(2, NP)


def _proj_body(x_ref, w_ref, b_ref, q0_ref, q1_ref, q2_ref, q3_ref):
    y = jax.lax.dot_general(x_ref[...], w_ref[0, 0],
                            (((1,), (1,)), ((), ())),
                            preferred_element_type=jnp.float32)
    y = y + b_ref[0, 0]
    q0_ref[...] = y[:, 0:96]
    q1_ref[...] = y[:, 96:192]
    q2_ref[...] = y[:, 192:288]
    q3_ref[...] = y[:, 288:384]


def _projections(x_upd, W_all, b_all):
    nr = NP // RB
    ospec = pl.BlockSpec((RB, 96), lambda b, s, r: ((b * 2 + s) * nr + r, 0))
    return pl.pallas_call(
        _proj_body,
        grid=(2, 2, nr),
        in_specs=[
            pl.BlockSpec((RB, CP), lambda b, s, r: (b * nr + r, 0)),
            pl.BlockSpec((1, 1, CP, CP), lambda b, s, r: (b, s, 0, 0)),
            pl.BlockSpec((1, 1, 1, CP), lambda b, s, r: (b, s, 0, 0)),
        ],
        out_specs=[ospec] * 4,
        out_shape=[jax.ShapeDtypeStruct((4 * NP, 96), jnp.float32)] * 4,
    )(x_upd, W_all, b_all)


def _final_body(q0_ref, q1_ref, q2_ref, q3_ref, den_ref, x_ref, bias_ref,
                out_ref):
    acc = jnp.concatenate(
        [q0_ref[...], q1_ref[...], q2_ref[...], q3_ref[...]], axis=1)
    den = den_ref[...].reshape(RB, 1)
    gat = jnp.where(den > 0.0, acc / den, 0.0) + bias_ref[0, 0]
    out_ref[...] = jnp.maximum(gat + x_ref[...], 0.0)


def _final_combine(acc, denom, x_upd, bias2):
    nr = NP // RB
    nh = 2 * NP // RB

    def qspec(q):
        return pl.BlockSpec((RB, 96), lambda b, r: (q * nh + b * nr + r, 0))

    return pl.pallas_call(
        _final_body,
        grid=(2, nr),
        in_specs=[
            qspec(0), qspec(1), qspec(2), qspec(3),
            pl.BlockSpec((RB,), lambda b, r: (b * nr + r,)),
            pl.BlockSpec((RB, CP), lambda b, r: (b * nr + r, 0)),
            pl.BlockSpec((1, 1, CP), lambda b, r: (b, 0, 0)),
        ],
        out_specs=pl.BlockSpec((RB, CP), lambda b, r: (b * nr + r, 0)),
        out_shape=jax.ShapeDtypeStruct((2 * NP, CP), jnp.float32),
    )(acc, acc, acc, acc, denom, x_upd, bias2)


def _sc_scatter_body(x_hbm, src_hbm, dst_hbm, es_hbm, nid_hbm, out_hbm,
                     nid_v, srcv, dstv, esv, pos1v, pos2v, cbuf, sem):
    c = lax.axis_index("c")
    t = lax.axis_index("s")
    half = NP * CP
    per_tile = half // 16
    ch = 49152
    base = c * half + t * per_tile

    def copy_chunk(i, _):
        pltpu.sync_copy(x_hbm.at[pl.ds(base + i * ch, ch)], cbuf)
        pltpu.sync_copy(cbuf, out_hbm.at[pl.ds(base + i * ch, ch)])
        return 0

    lax.fori_loop(0, per_tile // ch, copy_chunk, 0)
    plsc.subcore_barrier()

    pltpu.sync_copy(nid_hbm.at[pl.ds(c * NP, NP)], nid_v)
    row0 = c * (NROW // 2) + t * 80
    bflat = c * half
    for ci in range(5):
        r = row0 + ci * 16
        pltpu.sync_copy(src_hbm.at[pl.ds(r, 16)], srcv)
        pltpu.sync_copy(dst_hbm.at[pl.ds(r, 16)], dstv)
        pltpu.sync_copy(es_hbm.at[pl.ds(r, 16)], esv)

        def compute(tt, _):
            j = tt // 8
            sl = pl.ds((tt % 8) * 16, 16)
            s16 = srcv[j, sl]
            d16 = dstv[j, sl]
            sid = plsc.load_gather(nid_v, [s16])
            tid = plsc.load_gather(nid_v, [d16])
            pos1v[j, sl] = bflat + d16 * CP + (IN_IDX + sid)
            pos2v[j, sl] = bflat + s16 * CP + (OUT_IDX + tid)
            return 0

        lax.fori_loop(0, 128, compute, 0)
        handles = []
        for j in range(16):
            handles.append(
                pltpu.async_copy(esv.at[j], out_hbm.at[pos1v.at[j]], sem))
            handles.append(
                pltpu.async_copy(esv.at[j], out_hbm.at[pos2v.at[j]], sem))
        for h in handles:
            h.wait()


def _sc_scatter(x_flat, src2d, dst2d, es2d, nid_flat):
    i32 = jnp.int32
    f32 = jnp.float32
    mesh = plsc.VectorSubcoreMesh(core_axis_name="c", subcore_axis_name="s")
    f = pl.kernel(
        _sc_scatter_body,
        out_type=jax.ShapeDtypeStruct((2 * NP * CP,), f32),
        mesh=mesh,
        compiler_params=pltpu.CompilerParams(needs_layout_passes=False, use_tc_tiling_on_sc=False),
        scratch_types=[
            pltpu.VMEM((NP,), i32),
            pltpu.VMEM((16, 128), i32), pltpu.VMEM((16, 128), i32),
            pltpu.VMEM((16, 128), f32),
            pltpu.VMEM((16, 128), i32), pltpu.VMEM((16, 128), i32),
            pltpu.VMEM((49152,), f32),
            pltpu.SemaphoreType.DMA,
        ],
    )
    return f(x_flat, src2d, dst2d, es2d, nid_flat)


def _sc_edge_body(L_hbm, src_hbm, dst_hbm, att_hbm, p_hbm,
                  attv, srcv, dstv, gsa, gsb, gda, gdb,
                  ua, ub, va, vb, pb, prow, sem, sem2):
    c = lax.axis_index("c")
    t = lax.axis_index("s")
    iota16 = lax.broadcasted_iota(jnp.int32, (16,), 0)
    for b in range(2):
        pltpu.sync_copy(att_hbm.at[pl.ds(((2 * c) * 2 + b) * 96, 96)],
                        attv.at[pl.ds(0, 96)])
        pltpu.sync_copy(att_hbm.at[pl.ds(((2 * c + 1) * 2 + b) * 96, 96)],
                        attv.at[pl.ds(96, 96)])
        att_regs = [attv[pl.ds(16 * j, 16)] for j in range(12)]
        base_a = (2 * c) * 4 * NP + (b * 2) * NP
        base_b = (2 * c + 1) * 4 * NP + (b * 2) * NP

        def chunk(ci, _):
            row = b * 1280 + t * 80 + ci
            pltpu.sync_copy(src_hbm.at[pl.ds(row, 1)], srcv)
            pltpu.sync_copy(dst_hbm.at[pl.ds(row, 1)], dstv)

            def mkidx(m, _):
                sl = pl.ds(m * 16, 16)
                s16 = srcv[0, sl]
                d16 = dstv[0, sl]
                gsa[0, sl] = s16 + base_a
                gsb[0, sl] = s16 + base_b
                gda[0, sl] = d16 + (base_a + NP)
                gdb[0, sl] = d16 + (base_b + NP)
                return 0

            lax.fori_loop(0, 8, mkidx, 0)
            h1 = pltpu.async_copy(L_hbm.at[gsa.at[0]], ua, sem)
            h2 = pltpu.async_copy(L_hbm.at[gda.at[0]], va, sem2)
            h1.wait()
            h2.wait()
            h1 = pltpu.async_copy(L_hbm.at[gsb.at[0]], ub, sem)
            h2 = pltpu.async_copy(L_hbm.at[gdb.at[0]], vb, sem2)
            h1.wait()
            h2.wait()

            def edge(e, _):
                acc = jnp.zeros((16,), jnp.float32)
                for j in range(6):
                    sl = pl.ds(16 * j, 16)
                    m_ = ua[e, sl] + va[e, sl]
                    lr = jnp.maximum(m_, 0.0) + 0.2 * jnp.minimum(m_, 0.0)
                    acc = acc + att_regs[j] * lr
                    m2 = ub[e, sl] + vb[e, sl]
                    lr2 = jnp.maximum(m2, 0.0) + 0.2 * jnp.minimum(m2, 0.0)
                    acc = acc + att_regs[j + 6] * lr2
                pb[e, pl.ds(0, 16)] = acc
                return 0

            lax.fori_loop(0, 128, edge, 0)

            def group(g, _):
                tot = jnp.zeros((16,), jnp.float32)
                rows = g * 16 + iota16
                for l in range(16):
                    col = jnp.full((16,), l, jnp.int32)
                    tot = tot + plsc.load_gather(pb, [rows, col])
                prow[pl.ds(g * 16, 16)] = tot
                return 0

            lax.fori_loop(0, 8, group, 0)
            out_off = c * E_TOT + b * NEP + t * 10240 + ci * 128
            pltpu.sync_copy(prow, p_hbm.at[pl.ds(out_off, 128)])
            return 0

        lax.fori_loop(0, 80, chunk, 0)


def _sc_edge(L, src2d, dst2d, att_flat):
    i32 = jnp.int32
    f32 = jnp.float32
    mesh = plsc.VectorSubcoreMesh(core_axis_name="c", subcore_axis_name="s")
    f = pl.kernel(
        _sc_edge_body,
        out_type=jax.ShapeDtypeStruct((2 * E_TOT,), f32),
        mesh=mesh,
        compiler_params=pltpu.CompilerParams(needs_layout_passes=False, use_tc_tiling_on_sc=False),
        scratch_types=[
            pltpu.VMEM((192,), f32),
            pltpu.VMEM((1, 128), i32), pltpu.VMEM((1, 128), i32),
            pltpu.VMEM((1, 128), i32), pltpu.VMEM((1, 128), i32),
            pltpu.VMEM((1, 128), i32), pltpu.VMEM((1, 128), i32),
            pltpu.VMEM((128, 96), f32), pltpu.VMEM((128, 96), f32),
            pltpu.VMEM((128, 96), f32), pltpu.VMEM((128, 96), f32),
            pltpu.VMEM((128, 16), f32), pltpu.VMEM((128,), f32),
            pltpu.SemaphoreType.DMA, pltpu.SemaphoreType.DMA,
        ],
    )
    return f(L, src2d, dst2d, att_flat)


def _sc_accum_body(L_hbm, src_hbm, dst_hbm, p_hbm, acc_out, den_out,
                   srcv, dstv, gsv, p0v, p1v, exv, u_rows, zbuf, zden,
                   acc_s, den_s, sem):
    c = lax.axis_index("c")
    t = lax.axis_index("s")

    def zb(z, _):
        zbuf[z // 6, pl.ds(16 * (z % 6), 16)] = jnp.zeros((16,), jnp.float32)
        return 0

    lax.fori_loop(0, 128 * 6, zb, 0)

    def zd(z, _):
        zden[pl.ds(16 * z, 16)] = jnp.zeros((16,), jnp.float32)
        return 0

    lax.fori_loop(0, 40, zd, 0)

    for p_i in range(2):
        q = 2 * c + p_i
        for b in range(2):
            for z5 in range(5):
                pltpu.sync_copy(zbuf, acc_s.at[pl.ds(t * 640 + z5 * 128, 128)])

            if p_i == 0:
                @pl.when(c == 0)
                def _():
                    pltpu.sync_copy(zden, den_s.at[pl.ds(t * 640, 640)])

            plsc.subcore_barrier()
            base_u = q * 4 * NP + (b * 2) * NP

            def chunk(ci, _):
                row = b * 1280 + t * 80 + ci
                pltpu.sync_copy(src_hbm.at[pl.ds(row, 1)], srcv)
                pltpu.sync_copy(dst_hbm.at[pl.ds(row, 1)], dstv)
                epos = b * NEP + t * 10240 + ci * 128
                pltpu.sync_copy(p_hbm.at[pl.ds(epos, 128)], p0v)
                pltpu.sync_copy(p_hbm.at[pl.ds(E_TOT + epos, 128)], p1v)

                def mkidx(m, _):
                    sl = pl.ds(m * 16, 16)
                    gsv[0, sl] = srcv[0, sl] + base_u
                    exv[sl] = jnp.exp(p0v[sl] + p1v[sl])
                    return 0

                lax.fori_loop(0, 8, mkidx, 0)
                pltpu.async_copy(L_hbm.at[gsv.at[0]], u_rows, sem).wait()

                def egroup(g, _):
                    v16 = exv[pl.ds(16 * g, 16)]
                    for l in range(16):
                        bc = v16.at[jnp.full((16,), l, jnp.int32)].get(
                            mode='promise_in_bounds')
                        e = g * 16 + l
                        for j in range(6):
                            sl = pl.ds(16 * j, 16)
                            u_rows[e, sl] = u_rows[e, sl] * bc
                    return 0

                lax.fori_loop(0, 8, egroup, 0)
                pltpu.sync_copy(u_rows, acc_s.at[dstv.at[0]], add=True)

                if p_i == 0:
                    @pl.when(c == 0)
                    def _():
                        pltpu.sync_copy(exv, den_s.at[dstv.at[0]], add=True)

                return 0

            lax.fori_loop(0, 80, chunk, 0)
            plsc.subcore_barrier()
            pltpu.sync_copy(
                acc_s.at[pl.ds(t * 640, 640)],
                acc_out.at[pl.ds(q * 2 * NP + b * NP + t * 640, 640)])

            if p_i == 0:
                @pl.when(c == 0)
                def _():
                    pltpu.sync_copy(den_s.at[pl.ds(t * 640, 640)],
                                    den_out.at[pl.ds(b * NP + t * 640, 640)])


def _sc_accum(L, src2d, dst2d, p):
    i32 = jnp.int32
    f32 = jnp.float32
    mesh = plsc.VectorSubcoreMesh(core_axis_name="c", subcore_axis_name="s")
    f = pl.kernel(
        _sc_accum_body,
        out_type=(jax.ShapeDtypeStruct((8 * NP, 96), f32),
                  jax.ShapeDtypeStruct((2 * NP,), f32)),
        mesh=mesh,
        compiler_params=pltpu.CompilerParams(needs_layout_passes=False, use_tc_tiling_on_sc=False),
        scratch_types=[
            pltpu.VMEM((1, 128), i32), pltpu.VMEM((1, 128), i32),
            pltpu.VMEM((1, 128), i32),
            pltpu.VMEM((128,), f32), pltpu.VMEM((128,), f32),
            pltpu.VMEM((128,), f32),
            pltpu.VMEM((128, 96), f32), pltpu.VMEM((128, 96), f32),
            pltpu.VMEM((640,), f32),
            pltpu.VMEM_SHARED((NP, 96), f32), pltpu.VMEM_SHARED((NP,), f32),
            pltpu.SemaphoreType.DMA,
        ],
    )
    return f(L, src2d, dst2d, p)


def _pad2(a):
    return jnp.pad(a, ((0, CP - HID), (0, CP - HID)))


def _pad1(a):
    return jnp.pad(a, (0, CP - HID))


def kernel(fwd_x, fwd_edge_index, fwd_edge_attr, bwd_x, bwd_edge_index,
           bwd_edge_attr, embed, W_ih_f, W_hh_f, b_ih_f, b_hh_f, W_ih_r,
           W_hh_r, b_ih_r, b_hh_r, lin1_W, lin1_b, lin2_W, lin2_b, fc_Wl,
           fc_bl, fc_Wr, fc_br, fc_att, fc_bias, bc_Wl, bc_bl, bc_Wr, bc_br,
           bc_att, bc_bias):
    f32 = jnp.float32
    n = fwd_x.shape[0]

    tokens = jnp.concatenate(
        [jnp.pad(fwd_edge_attr, ((0, NEP - NE), (0, 0))),
         jnp.pad(bwd_edge_attr, ((0, NEP - NE), (0, 0)))],
        axis=0).astype(jnp.int32)
    e_scalar = _lstm_escalar(
        tokens, embed, W_ih_f.T, W_hh_f.T, (b_ih_f + b_hh_f).reshape(1, -1),
        W_ih_r.T, (b_ih_r + b_hh_r).reshape(1, -1), lin1_W.T,
        lin1_b.reshape(1, -1), lin2_W, lin2_b.reshape(1, 1))

    x2 = jnp.stack([jnp.pad(fwd_x, ((0, NP - n), (0, CP - HID))),
                    jnp.pad(bwd_x, ((0, NP - n), (0, CP - HID)))])
    nid = _node_argmax(x2[:, :, :128])

    src = jnp.stack([fwd_edge_index[0], bwd_edge_index[0]])
    dst = jnp.stack([fwd_edge_index[1], bwd_edge_index[1]])

    padv = (10000 + (jnp.arange(NEP - NE, dtype=jnp.int32) % 240))
    src2d = jnp.concatenate(
        [jnp.concatenate([fwd_edge_index[0].astype(jnp.int32), padv]),
         jnp.concatenate([bwd_edge_index[0].astype(jnp.int32), padv])]
    ).reshape(NROW, 128)
    dst2d = jnp.concatenate(
        [jnp.concatenate([fwd_edge_index[1].astype(jnp.int32), padv]),
         jnp.concatenate([bwd_edge_index[1].astype(jnp.int32), padv])]
    ).reshape(NROW, 128)
    es2d = e_scalar.reshape(NROW, 128)

    x_upd_flat = _sc_scatter(x2.reshape(2 * NP * CP), src2d, dst2d, es2d,
                             nid.reshape(2 * NP))
    x_upd = x_upd_flat.reshape(2 * NP, CP)

    W_all = jnp.stack([jnp.stack([_pad2(fc_Wl), _pad2(fc_Wr)]),
                       jnp.stack([_pad2(bc_Wl), _pad2(bc_Wr)])])
    b_all = jnp.stack([jnp.stack([_pad1(fc_bl), _pad1(fc_br)]),
                       jnp.stack([_pad1(bc_bl), _pad1(bc_br)])])
    quarters = _projections(x_upd, W_all, b_all.reshape(2, 2, 1, CP))

    att2 = jnp.stack([_pad1(fc_att), _pad1(bc_att)])
    att_flat = jnp.concatenate(
        [att2[b, q * 96:(q + 1) * 96] for q in range(4) for b in range(2)])

    L = jnp.concatenate(quarters, axis=0)
    p = _sc_edge(L, src2d, dst2d, att_flat)
    acc, denom = _sc_accum(L, src2d, dst2d, p)

    bias2 = jnp.stack([_pad1(fc_bias), _pad1(bc_bias)]).reshape(2, 1, CP)
    y = _final_combine(acc, denom, x_upd, bias2)
    return jnp.concatenate([y[:n, :HID], y[NP:NP + n, :HID]], axis=1)


# speedup vs baseline: 57.7528x; 1.7135x over previous
import functools

import jax
import jax.numpy as jnp
from jax import lax
from jax.experimental import pallas as pl
from jax.experimental.pallas import tpu as pltpu
from jax.experimental.pallas import tpu_sc as plsc

STATE_ID = 53
IN_IDX = 161
OUT_IDX = 320
HID = 373
NP = 10240
CP = 384
SEQ = 20
NE = 160000
NEP = 163840
E_TOT = 2 * NEP
E_BLK = 2048
RB = 1024
NROW = E_TOT // 128


def _lstm_body(tok_ref, embed_t_ref, wih_f_ref, whh_f_ref, b_f_ref,
               wih_r_ref, b_r_ref, lin1_ref, l1b_ref, lin2_ref,
               l2b_ref, out_ref):
    B = tok_ref.shape[1]
    H = 32
    f32 = jnp.float32
    bf16 = jnp.bfloat16
    ew_f = (jnp.dot(wih_f_ref[...], embed_t_ref[...],
                    preferred_element_type=f32) + b_f_ref[...]).astype(bf16)
    ew_r = (jnp.dot(wih_r_ref[...], embed_t_ref[...],
                    preferred_element_type=f32) + b_r_ref[...]).astype(bf16)
    whh = whh_f_ref[...].astype(bf16)
    iota_s = jax.lax.broadcasted_iota(jnp.int32, (128, B), 0)
    tok = tok_ref[...]

    def onehot_t(t):
        row = jax.lax.broadcast_in_dim(tok[t:t + 1, :], (128, B), (0, 1))
        return (row == iota_s).astype(bf16)

    def gates_act(g, c_prev):
        ii = g[0:H]
        ff = g[H:2 * H]
        gg = g[2 * H:3 * H]
        oo = g[3 * H:4 * H]
        c = jax.nn.sigmoid(ff) * c_prev + jax.nn.sigmoid(ii) * jnp.tanh(gg)
        h = jax.nn.sigmoid(oo) * jnp.tanh(c)
        return h, c

    h = jnp.zeros((H, B), f32)
    c = jnp.zeros((H, B), f32)
    for t in range(SEQ):
        g = jnp.dot(ew_f, onehot_t(t), preferred_element_type=f32)
        g = g + jnp.dot(whh, h.astype(bf16), preferred_element_type=f32)
        h, c = gates_act(g, c)
    g_r = jnp.dot(ew_r, onehot_t(SEQ - 1), preferred_element_type=f32)
    h_r, _ = gates_act(g_r, jnp.zeros((H, B), f32))

    feat = jnp.concatenate([h, h_r], axis=0).astype(bf16)
    e1 = jnp.maximum(
        jnp.dot(lin1_ref[...].astype(bf16), feat,
                preferred_element_type=f32) + l1b_ref[...], 0.0)
    es = jnp.dot(lin2_ref[...].astype(bf16), e1.astype(bf16),
                 preferred_element_type=f32) + l2b_ref[0, 0]
    out_ref[...] = jnp.maximum(es, 0.0).reshape(B)


def _lstm_escalar(tokens_t, embed_t, wih_f, whh_f, b_f, wih_r, b_r,
                  lin1_W, lin1_b, lin2_W, lin2_b):
    nblk = E_TOT // E_BLK

    def full(*shape):
        return pl.BlockSpec(shape, lambda i: (0,) * len(shape))

    return pl.pallas_call(
        _lstm_body,
        grid=(nblk,),
        in_specs=[
            pl.BlockSpec((SEQ, E_BLK), lambda i: (0, i)),
            full(32, 128),
            full(128, 32), full(128, 32), full(128, 1),
            full(128, 32), full(128, 1),
            full(32, 64), full(32, 1),
            full(1, 32), full(1, 1),
        ],
        out_specs=pl.BlockSpec((E_BLK,), lambda i: (i,)),
        out_shape=jax.ShapeDtypeStruct((E_TOT,), jnp.float32),
    )(tokens_t, embed_t, wih_f, whh_f, b_f, wih_r, b_r, lin1_W, lin1_b,
      lin2_W, lin2_b)


def _argmax_body(x_ref, out_ref):
    v = x_ref[0]
    iota_c = jax.lax.broadcasted_iota(jnp.int32, v.shape, 1)
    v = jnp.where(iota_c < STATE_ID, v, -1e30)
    m = jnp.max(v, axis=1, keepdims=True)
    idx = jnp.min(jnp.where(v >= m, iota_c, 128), axis=1)
    out_ref[...] = idx.reshape(1, 1, NP)


def _node_argmax(x2_head):
    out = pl.pallas_call(
        _argmax_body,
        grid=(2,),
        in_specs=[pl.BlockSpec((1, NP, 128), lambda b: (b, 0, 0))],
        out_specs=pl.BlockSpec((1, 1, NP), lambda b: (b, 0, 0)),
        out_shape=jax.ShapeDtypeStruct((2, 1, NP), jnp.int32),
    )(x2_head)
    return out.reshape(2, NP)


def _proj_body(x_ref, w_ref, b_ref, q0_ref, q1_ref, q2_ref, q3_ref):
    y = jax.lax.dot_general(x_ref[...], w_ref[0, 0],
                            (((1,), (1,)), ((), ())),
                            preferred_element_type=jnp.float32)
    y = y + b_ref[0, 0]
    q0_ref[...] = y[:, 0:96]
    q1_ref[...] = y[:, 96:192]
    q2_ref[...] = y[:, 192:288]
    q3_ref[...] = y[:, 288:384]


def _projections(x_upd, W_all, b_all):
    nr = NP // RB
    ospec = pl.BlockSpec((RB, 96), lambda b, s, r: ((b * 2 + s) * nr + r, 0))
    return pl.pallas_call(
        _proj_body,
        grid=(2, 2, nr),
        in_specs=[
            pl.BlockSpec((RB, CP), lambda b, s, r: (b * nr + r, 0)),
            pl.BlockSpec((1, 1, CP, CP), lambda b, s, r: (b, s, 0, 0)),
            pl.BlockSpec((1, 1, 1, CP), lambda b, s, r: (b, s, 0, 0)),
        ],
        out_specs=[ospec] * 4,
        out_shape=[jax.ShapeDtypeStruct((4 * NP, 96), jnp.float32)] * 4,
    )(x_upd, W_all, b_all)


def _final_body(q0_ref, q1_ref, q2_ref, q3_ref, den_ref, x_ref, bias_ref,
                out_ref):
    acc = jnp.concatenate(
        [q0_ref[...], q1_ref[...], q2_ref[...], q3_ref[...]], axis=1)
    den = den_ref[...].reshape(RB, 1)
    gat = jnp.where(den > 0.0, acc / den, 0.0) + bias_ref[0, 0]
    out_ref[...] = jnp.maximum(gat + x_ref[...], 0.0)


def _final_combine(acc, denom, x_upd, bias2):
    nr = NP // RB
    nh = 2 * NP // RB

    def qspec(q):
        return pl.BlockSpec((RB, 96), lambda b, r: (q * nh + b * nr + r, 0))

    return pl.pallas_call(
        _final_body,
        grid=(2, nr),
        in_specs=[
            qspec(0), qspec(1), qspec(2), qspec(3),
            pl.BlockSpec((RB,), lambda b, r: (b * nr + r,)),
            pl.BlockSpec((RB, CP), lambda b, r: (b * nr + r, 0)),
            pl.BlockSpec((1, 1, CP), lambda b, r: (b, 0, 0)),
        ],
        out_specs=pl.BlockSpec((RB, CP), lambda b, r: (b * nr + r, 0)),
        out_shape=jax.ShapeDtypeStruct((2 * NP, CP), jnp.float32),
    )(acc, acc, acc, acc, denom, x_upd, bias2)


def _sc_scatter_body(x_hbm, src_hbm, dst_hbm, es_hbm, nid_hbm, out_hbm,
                     nid_v, srcv, dstv, esv, pos1v, pos2v, cbuf, sem):
    c = lax.axis_index("c")
    t = lax.axis_index("s")
    half = NP * CP
    per_tile = half // 16
    ch = 49152
    base = c * half + t * per_tile

    def copy_chunk(i, _):
        pltpu.sync_copy(x_hbm.at[pl.ds(base + i * ch, ch)], cbuf)
        pltpu.sync_copy(cbuf, out_hbm.at[pl.ds(base + i * ch, ch)])
        return 0

    lax.fori_loop(0, per_tile // ch, copy_chunk, 0)
    plsc.subcore_barrier()

    pltpu.sync_copy(nid_hbm.at[pl.ds(c * NP, NP)], nid_v)
    row0 = c * (NROW // 2) + t * 80
    bflat = c * half
    for ci in range(5):
        r = row0 + ci * 16
        pltpu.sync_copy(src_hbm.at[pl.ds(r, 16)], srcv)
        pltpu.sync_copy(dst_hbm.at[pl.ds(r, 16)], dstv)
        pltpu.sync_copy(es_hbm.at[pl.ds(r, 16)], esv)

        def compute(tt, _):
            j = tt // 8
            sl = pl.ds((tt % 8) * 16, 16)
            s16 = srcv[j, sl]
            d16 = dstv[j, sl]
            sid = plsc.load_gather(nid_v, [s16])
            tid = plsc.load_gather(nid_v, [d16])
            pos1v[j, sl] = bflat + d16 * CP + (IN_IDX + sid)
            pos2v[j, sl] = bflat + s16 * CP + (OUT_IDX + tid)
            return 0

        lax.fori_loop(0, 128, compute, 0)
        handles = []
        for j in range(16):
            handles.append(
                pltpu.async_copy(esv.at[j], out_hbm.at[pos1v.at[j]], sem))
            handles.append(
                pltpu.async_copy(esv.at[j], out_hbm.at[pos2v.at[j]], sem))
        for h in handles:
            h.wait()


def _sc_scatter(x_flat, src2d, dst2d, es2d, nid_flat):
    i32 = jnp.int32
    f32 = jnp.float32
    mesh = plsc.VectorSubcoreMesh(core_axis_name="c", subcore_axis_name="s")
    f = pl.kernel(
        _sc_scatter_body,
        out_type=jax.ShapeDtypeStruct((2 * NP * CP,), f32),
        mesh=mesh,
        compiler_params=pltpu.CompilerParams(needs_layout_passes=False, use_tc_tiling_on_sc=False),
        scratch_types=[
            pltpu.VMEM((NP,), i32),
            pltpu.VMEM((16, 128), i32), pltpu.VMEM((16, 128), i32),
            pltpu.VMEM((16, 128), f32),
            pltpu.VMEM((16, 128), i32), pltpu.VMEM((16, 128), i32),
            pltpu.VMEM((49152,), f32),
            pltpu.SemaphoreType.DMA,
        ],
    )
    return f(x_flat, src2d, dst2d, es2d, nid_flat)


def _sc_edge_body(L_hbm, src_hbm, dst_hbm, att_hbm, p_hbm,
                  attv, srcv, dstv, gsa, gsb, gda, gdb,
                  ua, ub, va, vb, pb, prow, sem, sem2):
    c = lax.axis_index("c")
    t = lax.axis_index("s")
    iota16 = lax.broadcasted_iota(jnp.int32, (16,), 0)
    for b in range(2):
        pltpu.sync_copy(att_hbm.at[pl.ds(((2 * c) * 2 + b) * 96, 96)],
                        attv.at[pl.ds(0, 96)])
        pltpu.sync_copy(att_hbm.at[pl.ds(((2 * c + 1) * 2 + b) * 96, 96)],
                        attv.at[pl.ds(96, 96)])
        att_regs = [attv[pl.ds(16 * j, 16)] for j in range(12)]
        base_a = (2 * c) * 4 * NP + (b * 2) * NP
        base_b = (2 * c + 1) * 4 * NP + (b * 2) * NP

        def chunk(ci, _):
            row = b * 1280 + t * 80 + ci
            pltpu.sync_copy(src_hbm.at[pl.ds(row, 1)], srcv)
            pltpu.sync_copy(dst_hbm.at[pl.ds(row, 1)], dstv)

            def mkidx(m, _):
                sl = pl.ds(m * 16, 16)
                s16 = srcv[0, sl]
                d16 = dstv[0, sl]
                gsa[0, sl] = s16 + base_a
                gsb[0, sl] = s16 + base_b
                gda[0, sl] = d16 + (base_a + NP)
                gdb[0, sl] = d16 + (base_b + NP)
                return 0

            lax.fori_loop(0, 8, mkidx, 0)
            h1 = pltpu.async_copy(L_hbm.at[gsa.at[0]], ua, sem)
            h2 = pltpu.async_copy(L_hbm.at[gda.at[0]], va, sem2)
            h1.wait()
            h2.wait()
            h1 = pltpu.async_copy(L_hbm.at[gsb.at[0]], ub, sem)
            h2 = pltpu.async_copy(L_hbm.at[gdb.at[0]], vb, sem2)
            h1.wait()
            h2.wait()

            def edge(e, _):
                acc = jnp.zeros((16,), jnp.float32)
                for j in range(6):
                    sl = pl.ds(16 * j, 16)
                    m_ = ua[e, sl] + va[e, sl]
                    lr = jnp.maximum(m_, 0.0) + 0.2 * jnp.minimum(m_, 0.0)
                    acc = acc + att_regs[j] * lr
                    m2 = ub[e, sl] + vb[e, sl]
                    lr2 = jnp.maximum(m2, 0.0) + 0.2 * jnp.minimum(m2, 0.0)
                    acc = acc + att_regs[j + 6] * lr2
                pb[e, pl.ds(0, 16)] = acc
                return 0

            lax.fori_loop(0, 128, edge, 0)

            def group(g, _):
                tot = jnp.zeros((16,), jnp.float32)
                rows = g * 16 + iota16
                for l in range(16):
                    col = jnp.full((16,), l, jnp.int32)
                    tot = tot + plsc.load_gather(pb, [rows, col])
                prow[pl.ds(g * 16, 16)] = tot
                return 0

            lax.fori_loop(0, 8, group, 0)
            out_off = c * E_TOT + b * NEP + t * 10240 + ci * 128
            pltpu.sync_copy(prow, p_hbm.at[pl.ds(out_off, 128)])
            return 0

        lax.fori_loop(0, 80, chunk, 0)


def _sc_edge(L, src2d, dst2d, att_flat):
    i32 = jnp.int32
    f32 = jnp.float32
    mesh = plsc.VectorSubcoreMesh(core_axis_name="c", subcore_axis_name="s")
    f = pl.kernel(
        _sc_edge_body,
        out_type=jax.ShapeDtypeStruct((2 * E_TOT,), f32),
        mesh=mesh,
        compiler_params=pltpu.CompilerParams(needs_layout_passes=False, use_tc_tiling_on_sc=False),
        scratch_types=[
            pltpu.VMEM((192,), f32),
            pltpu.VMEM((1, 128), i32), pltpu.VMEM((1, 128), i32),
            pltpu.VMEM((1, 128), i32), pltpu.VMEM((1, 128), i32),
            pltpu.VMEM((1, 128), i32), pltpu.VMEM((1, 128), i32),
            pltpu.VMEM((128, 96), f32), pltpu.VMEM((128, 96), f32),
            pltpu.VMEM((128, 96), f32), pltpu.VMEM((128, 96), f32),
            pltpu.VMEM((128, 16), f32), pltpu.VMEM((128,), f32),
            pltpu.SemaphoreType.DMA, pltpu.SemaphoreType.DMA,
        ],
    )
    return f(L, src2d, dst2d, att_flat)


def _sc_accum_body(L_hbm, src_hbm, dst_hbm, p_hbm, acc_out, den_out,
                   srcv, dstv, gsv, p0v, p1v, exv, u_rows, zbuf, zden,
                   acc_s, den_s, sem):
    c = lax.axis_index("c")
    t = lax.axis_index("s")

    def zb(z, _):
        zbuf[z // 6, pl.ds(16 * (z % 6), 16)] = jnp.zeros((16,), jnp.float32)
        return 0

    lax.fori_loop(0, 128 * 6, zb, 0)

    def zd(z, _):
        zden[pl.ds(16 * z, 16)] = jnp.zeros((16,), jnp.float32)
        return 0

    lax.fori_loop(0, 40, zd, 0)

    for p_i in range(2):
        q = 2 * c + p_i
        for b in range(2):
            for z5 in range(5):
                pltpu.sync_copy(zbuf, acc_s.at[pl.ds(t * 640 + z5 * 128, 128)])

            if p_i == 0:
                @pl.when(c == 0)
                def _():
                    pltpu.sync_copy(zden, den_s.at[pl.ds(t * 640, 640)])

            plsc.subcore_barrier()
            base_u = q * 4 * NP + (b * 2) * NP

            def chunk(ci, _):
                row = b * 1280 + t * 80 + ci
                pltpu.sync_copy(src_hbm.at[pl.ds(row, 1)], srcv)
                pltpu.sync_copy(dst_hbm.at[pl.ds(row, 1)], dstv)
                epos = b * NEP + t * 10240 + ci * 128
                pltpu.sync_copy(p_hbm.at[pl.ds(epos, 128)], p0v)
                pltpu.sync_copy(p_hbm.at[pl.ds(E_TOT + epos, 128)], p1v)

                def mkidx(m, _):
                    sl = pl.ds(m * 16, 16)
                    gsv[0, sl] = srcv[0, sl] + base_u
                    exv[sl] = jnp.exp(p0v[sl] + p1v[sl])
                    return 0

                lax.fori_loop(0, 8, mkidx, 0)
                pltpu.async_copy(L_hbm.at[gsv.at[0]], u_rows, sem).wait()

                def egroup(g, _):
                    v16 = exv[pl.ds(16 * g, 16)]
                    for l in range(16):
                        bc = v16.at[jnp.full((16,), l, jnp.int32)].get(
                            mode='promise_in_bounds')
                        e = g * 16 + l
                        for j in range(6):
                            sl = pl.ds(16 * j, 16)
                            u_rows[e, sl] = u_rows[e, sl] * bc
                    return 0

                lax.fori_loop(0, 8, egroup, 0)
                pltpu.sync_copy(u_rows, acc_s.at[dstv.at[0]], add=True)

                if p_i == 0:
                    @pl.when(c == 0)
                    def _():
                        pltpu.sync_copy(exv, den_s.at[dstv.at[0]], add=True)

                return 0

            lax.fori_loop(0, 80, chunk, 0)
            plsc.subcore_barrier()
            pltpu.sync_copy(
                acc_s.at[pl.ds(t * 640, 640)],
                acc_out.at[pl.ds(q * 2 * NP + b * NP + t * 640, 640)])

            if p_i == 0:
                @pl.when(c == 0)
                def _():
                    pltpu.sync_copy(den_s.at[pl.ds(t * 640, 640)],
                                    den_out.at[pl.ds(b * NP + t * 640, 640)])


def _sc_accum(L, src2d, dst2d, p):
    i32 = jnp.int32
    f32 = jnp.float32
    mesh = plsc.VectorSubcoreMesh(core_axis_name="c", subcore_axis_name="s")
    f = pl.kernel(
        _sc_accum_body,
        out_type=(jax.ShapeDtypeStruct((8 * NP, 96), f32),
                  jax.ShapeDtypeStruct((2 * NP,), f32)),
        mesh=mesh,
        compiler_params=pltpu.CompilerParams(needs_layout_passes=False, use_tc_tiling_on_sc=False),
        scratch_types=[
            pltpu.VMEM((1, 128), i32), pltpu.VMEM((1, 128), i32),
            pltpu.VMEM((1, 128), i32),
            pltpu.VMEM((128,), f32), pltpu.VMEM((128,), f32),
            pltpu.VMEM((128,), f32),
            pltpu.VMEM((128, 96), f32), pltpu.VMEM((128, 96), f32),
            pltpu.VMEM((640,), f32),
            pltpu.VMEM_SHARED((NP, 96), f32), pltpu.VMEM_SHARED((NP,), f32),
            pltpu.SemaphoreType.DMA,
        ],
    )
    return f(L, src2d, dst2d, p)


def _pad2(a):
    return jnp.pad(a, ((0, CP - HID), (0, CP - HID)))


def _pad1(a):
    return jnp.pad(a, (0, CP - HID))


def kernel(fwd_x, fwd_edge_index, fwd_edge_attr, bwd_x, bwd_edge_index,
           bwd_edge_attr, embed, W_ih_f, W_hh_f, b_ih_f, b_hh_f, W_ih_r,
           W_hh_r, b_ih_r, b_hh_r, lin1_W, lin1_b, lin2_W, lin2_b, fc_Wl,
           fc_bl, fc_Wr, fc_br, fc_att, fc_bias, bc_Wl, bc_bl, bc_Wr, bc_br,
           bc_att, bc_bias):
    f32 = jnp.float32
    n = fwd_x.shape[0]

    tokens_t = jnp.concatenate(
        [jnp.pad(fwd_edge_attr, ((0, NEP - NE), (0, 0))),
         jnp.pad(bwd_edge_attr, ((0, NEP - NE), (0, 0)))],
        axis=0).astype(jnp.int32).T
    e_scalar = _lstm_escalar(
        tokens_t, embed.T, W_ih_f, W_hh_f, (b_ih_f + b_hh_f).reshape(-1, 1),
        W_ih_r, (b_ih_r + b_hh_r).reshape(-1, 1), lin1_W,
        lin1_b.reshape(-1, 1), lin2_W, lin2_b.reshape(1, 1))

    x2 = jnp.stack([jnp.pad(fwd_x, ((0, NP - n), (0, CP - HID))),
                    jnp.pad(bwd_x, ((0, NP - n), (0, CP - HID)))])
    nid = _node_argmax(x2[:, :, :128])

    src = jnp.stack([fwd_edge_index[0], bwd_edge_index[0]])
    dst = jnp.stack([fwd_edge_index[1], bwd_edge_index[1]])

    padv = (10000 + (jnp.arange(NEP - NE, dtype=jnp.int32) % 240))
    src2d = jnp.concatenate(
        [jnp.concatenate([fwd_edge_index[0].astype(jnp.int32), padv]),
         jnp.concatenate([bwd_edge_index[0].astype(jnp.int32), padv])]
    ).reshape(NROW, 128)
    dst2d = jnp.concatenate(
        [jnp.concatenate([fwd_edge_index[1].astype(jnp.int32), padv]),
         jnp.concatenate([bwd_edge_index[1].astype(jnp.int32), padv])]
    ).reshape(NROW, 128)
    es2d = e_scalar.reshape(NROW, 128)

    x_upd_flat = _sc_scatter(x2.reshape(2 * NP * CP), src2d, dst2d, es2d,
                             nid.reshape(2 * NP))
    x_upd = x_upd_flat.reshape(2 * NP, CP)

    W_all = jnp.stack([jnp.stack([_pad2(fc_Wl), _pad2(fc_Wr)]),
                       jnp.stack([_pad2(bc_Wl), _pad2(bc_Wr)])])
    b_all = jnp.stack([jnp.stack([_pad1(fc_bl), _pad1(fc_br)]),
                       jnp.stack([_pad1(bc_bl), _pad1(bc_br)])])
    quarters = _projections(x_upd, W_all, b_all.reshape(2, 2, 1, CP))

    att2 = jnp.stack([_pad1(fc_att), _pad1(bc_att)])
    att_flat = jnp.concatenate(
        [att2[b, q * 96:(q + 1) * 96] for q in range(4) for b in range(2)])

    L = jnp.concatenate(quarters, axis=0)
    p = _sc_edge(L, src2d, dst2d, att_flat)
    acc, denom = _sc_accum(L, src2d, dst2d, p)

    bias2 = jnp.stack([_pad1(fc_bias), _pad1(bc_bias)]).reshape(2, 1, CP)
    y = _final_combine(acc, denom, x_upd, bias2)
    return jnp.concatenate([y[:n, :HID], y[NP:NP + n, :HID]], axis=1)


# speedup vs baseline: 65.1689x; 1.1284x over previous
import functools

import jax
import jax.numpy as jnp
from jax import lax
from jax.experimental import pallas as pl
from jax.experimental.pallas import tpu as pltpu
from jax.experimental.pallas import tpu_sc as plsc

STATE_ID = 53
IN_IDX = 161
OUT_IDX = 320
HID = 373
NP = 10240
CP = 384
SEQ = 20
NE = 160000
NEP = 163840
E_TOT = 2 * NEP
E_BLK = 2048
RB = 1024
NROW = E_TOT // 128


def _lstm_body(tok_ref, embed_t_ref, wih_f_ref, whh_f_ref, b_f_ref,
               wih_r_ref, b_r_ref, lin1_ref, l1b_ref, lin2_ref,
               l2b_ref, out_ref):
    B = tok_ref.shape[1]
    H = 32
    f32 = jnp.float32
    bf16 = jnp.bfloat16
    ew_f = (jnp.dot(wih_f_ref[...], embed_t_ref[...],
                    preferred_element_type=f32) + b_f_ref[...]).astype(bf16)
    ew_r = (jnp.dot(wih_r_ref[...], embed_t_ref[...],
                    preferred_element_type=f32) + b_r_ref[...]).astype(bf16)
    whh = whh_f_ref[...].astype(bf16)
    iota_s = jax.lax.broadcasted_iota(jnp.int32, (128, B), 0)
    tok = tok_ref[...]

    def onehot_t(t):
        row = jax.lax.broadcast_in_dim(tok[t:t + 1, :], (128, B), (0, 1))
        return (row == iota_s).astype(bf16)

    def gates_act(g, c_prev):
        ii = g[0:H]
        ff = g[H:2 * H]
        gg = g[2 * H:3 * H]
        oo = g[3 * H:4 * H]
        c = jax.nn.sigmoid(ff) * c_prev + jax.nn.sigmoid(ii) * jnp.tanh(gg)
        h = jax.nn.sigmoid(oo) * jnp.tanh(c)
        return h, c

    h = jnp.zeros((H, B), f32)
    c = jnp.zeros((H, B), f32)
    for t in range(SEQ):
        g = jnp.dot(ew_f, onehot_t(t), preferred_element_type=f32)
        g = g + jnp.dot(whh, h.astype(bf16), preferred_element_type=f32)
        h, c = gates_act(g, c)
    g_r = jnp.dot(ew_r, onehot_t(SEQ - 1), preferred_element_type=f32)
    h_r, _ = gates_act(g_r, jnp.zeros((H, B), f32))

    feat = jnp.concatenate([h, h_r], axis=0).astype(bf16)
    e1 = jnp.maximum(
        jnp.dot(lin1_ref[...].astype(bf16), feat,
                preferred_element_type=f32) + l1b_ref[...], 0.0)
    es = jnp.dot(lin2_ref[...].astype(bf16), e1.astype(bf16),
                 preferred_element_type=f32) + l2b_ref[0, 0]
    out_ref[...] = jnp.maximum(es, 0.0).reshape(B)


def _lstm_escalar(tokens_t, embed_t, wih_f, whh_f, b_f, wih_r, b_r,
                  lin1_W, lin1_b, lin2_W, lin2_b):
    nblk = E_TOT // E_BLK

    def full(*shape):
        return pl.BlockSpec(shape, lambda i: (0,) * len(shape))

    return pl.pallas_call(
        _lstm_body,
        grid=(nblk,),
        in_specs=[
            pl.BlockSpec((SEQ, E_BLK), lambda i: (0, i)),
            full(32, 128),
            full(128, 32), full(128, 32), full(128, 1),
            full(128, 32), full(128, 1),
            full(32, 64), full(32, 1),
            full(1, 32), full(1, 1),
        ],
        out_specs=pl.BlockSpec((E_BLK,), lambda i: (i,)),
        out_shape=jax.ShapeDtypeStruct((E_TOT,), jnp.float32),
    )(tokens_t, embed_t, wih_f, whh_f, b_f, wih_r, b_r, lin1_W, lin1_b,
      lin2_W, lin2_b)


def _argmax_body(x_ref, out_ref):
    v = x_ref[0]
    iota_c = jax.lax.broadcasted_iota(jnp.int32, v.shape, 1)
    v = jnp.where(iota_c < STATE_ID, v, -1e30)
    m = jnp.max(v, axis=1, keepdims=True)
    idx = jnp.min(jnp.where(v >= m, iota_c, 128), axis=1)
    out_ref[...] = idx.reshape(1, 1, NP)


def _node_argmax(x2_head):
    out = pl.pallas_call(
        _argmax_body,
        grid=(2,),
        in_specs=[pl.BlockSpec((1, NP, 128), lambda b: (b, 0, 0))],
        out_specs=pl.BlockSpec((1, 1, NP), lambda b: (b, 0, 0)),
        out_shape=jax.ShapeDtypeStruct((2, 1, NP), jnp.int32),
    )(x2_head)
    return out.reshape(2, NP)


def _proj_body(x_ref, w_ref, b_ref, q0_ref, q1_ref, q2_ref, q3_ref):
    y = jax.lax.dot_general(x_ref[...], w_ref[0, 0],
                            (((1,), (1,)), ((), ())),
                            preferred_element_type=jnp.float32)
    y = y + b_ref[0, 0]
    q0_ref[...] = y[:, 0:96]
    q1_ref[...] = y[:, 96:192]
    q2_ref[...] = y[:, 192:288]
    q3_ref[...] = y[:, 288:384]


def _projections(x_upd, W_all, b_all):
    nr = NP // RB
    ospec = pl.BlockSpec((RB, 96), lambda b, s, r: ((b * 2 + s) * nr + r, 0))
    return pl.pallas_call(
        _proj_body,
        grid=(2, 2, nr),
        in_specs=[
            pl.BlockSpec((RB, CP), lambda b, s, r: (b * nr + r, 0)),
            pl.BlockSpec((1, 1, CP, CP), lambda b, s, r: (b, s, 0, 0)),
            pl.BlockSpec((1, 1, 1, CP), lambda b, s, r: (b, s, 0, 0)),
        ],
        out_specs=[ospec] * 4,
        out_shape=[jax.ShapeDtypeStruct((4 * NP, 96), jnp.float32)] * 4,
    )(x_upd, W_all, b_all)


def _final_body(q0_ref, q1_ref, q2_ref, q3_ref, den_ref, x_ref, bias_ref,
                out_ref):
    acc = jnp.concatenate(
        [q0_ref[...], q1_ref[...], q2_ref[...], q3_ref[...]], axis=1)
    den = den_ref[...].reshape(RB, 1)
    gat = jnp.where(den > 0.0, acc / den, 0.0) + bias_ref[0, 0]
    out_ref[...] = jnp.maximum(gat + x_ref[...], 0.0)


def _final_combine(acc, denom, x_upd, bias2):
    nr = NP // RB
    nh = 2 * NP // RB

    def qspec(q):
        return pl.BlockSpec((RB, 96), lambda b, r: (q * nh + b * nr + r, 0))

    return pl.pallas_call(
        _final_body,
        grid=(2, nr),
        in_specs=[
            qspec(0), qspec(1), qspec(2), qspec(3),
            pl.BlockSpec((RB,), lambda b, r: (b * nr + r,)),
            pl.BlockSpec((RB, CP), lambda b, r: (b * nr + r, 0)),
            pl.BlockSpec((1, 1, CP), lambda b, r: (b, 0, 0)),
        ],
        out_specs=pl.BlockSpec((RB, CP), lambda b, r: (b * nr + r, 0)),
        out_shape=jax.ShapeDtypeStruct((2 * NP, CP), jnp.float32),
    )(acc, acc, acc, acc, denom, x_upd, bias2)


def _sc_scatter_body(x_hbm, src_hbm, dst_hbm, es_hbm, nid_hbm, out_hbm,
                     nid_v, srcv, dstv, esv, pos1v, pos2v, cbufa, cbufb,
                     sem, semw):
    c = lax.axis_index("c")
    t = lax.axis_index("s")
    half = NP * CP
    per_tile = half // 16
    ch = 24576
    nch = per_tile // ch
    base = c * half + t * per_tile

    bufs = (cbufa, cbufb)
    rd = [None, None]
    wr = [None, None]
    rd[0] = pltpu.async_copy(x_hbm.at[pl.ds(base, ch)], cbufa, sem)
    for i in range(nch):
        cur = i % 2
        nxt = 1 - cur
        rd[cur].wait()
        if i + 1 < nch:
            if wr[nxt] is not None:
                wr[nxt].wait()
            rd[nxt] = pltpu.async_copy(
                x_hbm.at[pl.ds(base + (i + 1) * ch, ch)], bufs[nxt], sem)
        wr[cur] = pltpu.async_copy(
            bufs[cur], out_hbm.at[pl.ds(base + i * ch, ch)], semw)
    wr[0].wait()
    wr[1].wait()
    plsc.subcore_barrier()

    pltpu.sync_copy(nid_hbm.at[pl.ds(c * NP, NP)], nid_v)
    row0 = c * (NROW // 2) + t * 80
    bflat = c * half
    pltpu.sync_copy(src_hbm.at[pl.ds(row0, 80)], srcv)
    pltpu.sync_copy(dst_hbm.at[pl.ds(row0, 80)], dstv)
    pltpu.sync_copy(es_hbm.at[pl.ds(row0, 80)], esv)
    for ci in range(5):

        def compute(tt, _):
            j = tt // 8
            sl = pl.ds((tt % 8) * 16, 16)
            s16 = srcv[ci * 16 + j, sl]
            d16 = dstv[ci * 16 + j, sl]
            sid = plsc.load_gather(nid_v, [s16])
            tid = plsc.load_gather(nid_v, [d16])
            pos1v[j, sl] = bflat + d16 * CP + (IN_IDX + sid)
            pos2v[j, sl] = bflat + s16 * CP + (OUT_IDX + tid)
            return 0

        lax.fori_loop(0, 128, compute, 0)
        handles = []
        for j in range(16):
            handles.append(
                pltpu.async_copy(esv.at[ci * 16 + j],
                                 out_hbm.at[pos1v.at[j]], sem))
            handles.append(
                pltpu.async_copy(esv.at[ci * 16 + j],
                                 out_hbm.at[pos2v.at[j]], sem))
        for h in handles:
            h.wait()


def _sc_scatter(x_flat, src2d, dst2d, es2d, nid_flat):
    i32 = jnp.int32
    f32 = jnp.float32
    mesh = plsc.VectorSubcoreMesh(core_axis_name="c", subcore_axis_name="s")
    f = pl.kernel(
        _sc_scatter_body,
        out_type=jax.ShapeDtypeStruct((2 * NP * CP,), f32),
        mesh=mesh,
        compiler_params=pltpu.CompilerParams(needs_layout_passes=False, use_tc_tiling_on_sc=False),
        scratch_types=[
            pltpu.VMEM((NP,), i32),
            pltpu.VMEM((80, 128), i32), pltpu.VMEM((80, 128), i32),
            pltpu.VMEM((80, 128), f32),
            pltpu.VMEM((16, 128), i32), pltpu.VMEM((16, 128), i32),
            pltpu.VMEM((24576,), f32), pltpu.VMEM((24576,), f32),
            pltpu.SemaphoreType.DMA, pltpu.SemaphoreType.DMA,
        ],
    )
    return f(x_flat, src2d, dst2d, es2d, nid_flat)


def _sc_edge_body(L_hbm, src_hbm, dst_hbm, att_hbm, p_hbm,
                  attv, srcb, dstb, gsv, gdv, u, v, pb, prow, sem, sem2):
    c = lax.axis_index("c")
    t = lax.axis_index("s")
    iota16 = lax.broadcasted_iota(jnp.int32, (16,), 0)
    for b in range(2):
        pltpu.sync_copy(att_hbm.at[pl.ds(c * 384 + b * 192, 192)], attv)
        pltpu.sync_copy(src_hbm.at[pl.ds(b * 1280 + t * 80, 80)], srcb)
        pltpu.sync_copy(dst_hbm.at[pl.ds(b * 1280 + t * 80, 80)], dstb)
        att_regs = [attv[pl.ds(16 * j, 16)] for j in range(12)]
        base_u = c * 4 * NP + (b * 2) * NP
        base_v = base_u + NP

        def chunk(ci, _):
            def mkidx(m, _):
                sl = pl.ds(m * 16, 16)
                gsv[0, sl] = srcb[ci, sl] + base_u
                gdv[0, sl] = dstb[ci, sl] + base_v
                return 0

            lax.fori_loop(0, 8, mkidx, 0)
            h1 = pltpu.async_copy(L_hbm.at[gsv.at[0]], u, sem)
            h2 = pltpu.async_copy(L_hbm.at[gdv.at[0]], v, sem2)
            h1.wait()
            h2.wait()

            def edge(e, _):
                acc = jnp.zeros((16,), jnp.float32)
                for j in range(12):
                    sl = pl.ds(16 * j, 16)
                    m_ = u[e, sl] + v[e, sl]
                    lr = jnp.maximum(m_, 0.0) + 0.2 * jnp.minimum(m_, 0.0)
                    acc = acc + att_regs[j] * lr
                pb[e, pl.ds(0, 16)] = acc
                return 0

            lax.fori_loop(0, 128, edge, 0)

            def group(g, _):
                tot = jnp.zeros((16,), jnp.float32)
                rows = g * 16 + iota16
                for l in range(16):
                    col = jnp.full((16,), l, jnp.int32)
                    tot = tot + plsc.load_gather(pb, [rows, col])
                prow[0, pl.ds(g * 16, 16)] = tot
                return 0

            lax.fori_loop(0, 8, group, 0)
            out_off = c * E_TOT + b * NEP + t * 10240 + ci * 128
            pltpu.sync_copy(prow.at[0], p_hbm.at[pl.ds(out_off, 128)])
            return 0

        lax.fori_loop(0, 80, chunk, 0)


def _sc_edge(L, src2d, dst2d, att_flat):
    i32 = jnp.int32
    f32 = jnp.float32
    mesh = plsc.VectorSubcoreMesh(core_axis_name="c", subcore_axis_name="s")
    f = pl.kernel(
        _sc_edge_body,
        out_type=jax.ShapeDtypeStruct((2 * E_TOT,), f32),
        mesh=mesh,
        compiler_params=pltpu.CompilerParams(needs_layout_passes=False, use_tc_tiling_on_sc=False),
        scratch_types=[
            pltpu.VMEM((192,), f32),
            pltpu.VMEM((80, 128), i32), pltpu.VMEM((80, 128), i32),
            pltpu.VMEM((1, 128), i32), pltpu.VMEM((1, 128), i32),
            pltpu.VMEM((128, 192), f32), pltpu.VMEM((128, 192), f32),
            pltpu.VMEM((128, 16), f32), pltpu.VMEM((1, 128), f32),
            pltpu.SemaphoreType.DMA, pltpu.SemaphoreType.DMA,
        ],
    )
    return f(L, src2d, dst2d, att_flat)


def _sc_accum_body(L_hbm, src_hbm, dst_hbm, p_hbm, acc_out, den_out,
                   srcb, dstb, gsa, gsb, p0b, p1b, exa, exbb, ua, ubb,
                   zbuf, zden, acc_s, den_s, semga, semgb):
    c = lax.axis_index("c")
    t = lax.axis_index("s")

    def zb(z, _):
        zbuf[z // 6, pl.ds(16 * (z % 6), 16)] = jnp.zeros((16,), jnp.float32)
        return 0

    lax.fori_loop(0, 128 * 6, zb, 0)

    def zd(z, _):
        zden[pl.ds(16 * z, 16)] = jnp.zeros((16,), jnp.float32)
        return 0

    lax.fori_loop(0, 40, zd, 0)

    for p_i in range(2):
        q = 2 * c + p_i
        for b in range(2):
            for z5 in range(5):
                pltpu.sync_copy(zbuf, acc_s.at[pl.ds(t * 640 + z5 * 128, 128)])

            if p_i == 0:
                @pl.when(c == 0)
                def _():
                    pltpu.sync_copy(zden, den_s.at[pl.ds(t * 640, 640)])

            plsc.subcore_barrier()
            base_u = q * 4 * NP + (b * 2) * NP
            erow = b * 1280 + t * 80
            pltpu.sync_copy(src_hbm.at[pl.ds(erow, 80)], srcb)
            pltpu.sync_copy(dst_hbm.at[pl.ds(erow, 80)], dstb)
            epos = b * NEP + t * 10240

            gs = (gsa, gsb)
            ubufs = (ua, ubb)
            exbufs = (exa, exbb)
            sems = (semga, semgb)

            def mkidx(ci, which):
                def step(m, _):
                    sl = pl.ds(m * 16, 16)
                    gs[which][0, sl] = srcb[ci, sl] + base_u
                    return 0

                lax.fori_loop(0, 8, step, 0)

            def fire_gather(which):
                return pltpu.async_copy(L_hbm.at[gs[which].at[0]],
                                        ubufs[which], sems[which])

            def compute(ci, which):
                uw = ubufs[which]
                exw = exbufs[which]
                pltpu.sync_copy(p_hbm.at[pl.ds(epos + ci * 128, 128)], p0b)
                pltpu.sync_copy(p_hbm.at[pl.ds(E_TOT + epos + ci * 128, 128)],
                                p1b)

                def ex_step(m, _):
                    sl = pl.ds(m * 16, 16)
                    exw[sl] = jnp.exp(p0b[sl] + p1b[sl])
                    return 0

                lax.fori_loop(0, 8, ex_step, 0)

                def egroup(g, _):
                    v16 = exw[pl.ds(16 * g, 16)]
                    for l in range(16):
                        bc = v16.at[jnp.full((16,), l, jnp.int32)].get(
                            mode='promise_in_bounds')
                        e = g * 16 + l
                        for j in range(6):
                            sl = pl.ds(16 * j, 16)
                            uw[e, sl] = uw[e, sl] * bc
                    return 0

                lax.fori_loop(0, 8, egroup, 0)

            def scatter(ci, which):
                pltpu.sync_copy(ubufs[which], acc_s.at[dstb.at[ci]],
                                add=True)
                if p_i == 0:
                    @pl.when(c == 0)
                    def _():
                        pltpu.sync_copy(exbufs[which],
                                        den_s.at[dstb.at[ci]], add=True)

            def pair(i, _):
                mkidx(2 * i, 0)
                ha = fire_gather(0)
                mkidx(2 * i + 1, 1)
                ha.wait()
                hb = fire_gather(1)
                compute(2 * i, 0)
                scatter(2 * i, 0)
                hb.wait()
                compute(2 * i + 1, 1)
                scatter(2 * i + 1, 1)
                return 0

            lax.fori_loop(0, 40, pair, 0)
            plsc.subcore_barrier()
            pltpu.sync_copy(
                acc_s.at[pl.ds(t * 640, 640)],
                acc_out.at[pl.ds(q * 2 * NP + b * NP + t * 640, 640)])

            if p_i == 0:
                @pl.when(c == 0)
                def _():
                    pltpu.sync_copy(den_s.at[pl.ds(t * 640, 640)],
                                    den_out.at[pl.ds(b * NP + t * 640, 640)])


def _sc_accum(L, src2d, dst2d, p):
    i32 = jnp.int32
    f32 = jnp.float32
    mesh = plsc.VectorSubcoreMesh(core_axis_name="c", subcore_axis_name="s")
    f = pl.kernel(
        _sc_accum_body,
        out_type=(jax.ShapeDtypeStruct((8 * NP, 96), f32),
                  jax.ShapeDtypeStruct((2 * NP,), f32)),
        mesh=mesh,
        compiler_params=pltpu.CompilerParams(needs_layout_passes=False, use_tc_tiling_on_sc=False),
        scratch_types=[
            pltpu.VMEM((80, 128), i32), pltpu.VMEM((80, 128), i32),
            pltpu.VMEM((1, 128), i32), pltpu.VMEM((1, 128), i32),
            pltpu.VMEM((128,), f32), pltpu.VMEM((128,), f32),
            pltpu.VMEM((128,), f32), pltpu.VMEM((128,), f32),
            pltpu.VMEM((128, 96), f32), pltpu.VMEM((128, 96), f32),
            pltpu.VMEM((128, 96), f32), pltpu.VMEM((640,), f32),
            pltpu.VMEM_SHARED((NP, 96), f32), pltpu.VMEM_SHARED((NP,), f32),
            pltpu.SemaphoreType.DMA, pltpu.SemaphoreType.DMA,
        ],
    )
    return f(L, src2d, dst2d, p)


def _pad2(a):
    return jnp.pad(a, ((0, CP - HID), (0, CP - HID)))


def _pad1(a):
    return jnp.pad(a, (0, CP - HID))


def kernel(fwd_x, fwd_edge_index, fwd_edge_attr, bwd_x, bwd_edge_index,
           bwd_edge_attr, embed, W_ih_f, W_hh_f, b_ih_f, b_hh_f, W_ih_r,
           W_hh_r, b_ih_r, b_hh_r, lin1_W, lin1_b, lin2_W, lin2_b, fc_Wl,
           fc_bl, fc_Wr, fc_br, fc_att, fc_bias, bc_Wl, bc_bl, bc_Wr, bc_br,
           bc_att, bc_bias):
    f32 = jnp.float32
    n = fwd_x.shape[0]

    tokens_t = jnp.concatenate(
        [jnp.pad(fwd_edge_attr, ((0, NEP - NE), (0, 0))),
         jnp.pad(bwd_edge_attr, ((0, NEP - NE), (0, 0)))],
        axis=0).astype(jnp.int32).T
    e_scalar = _lstm_escalar(
        tokens_t, embed.T, W_ih_f, W_hh_f, (b_ih_f + b_hh_f).reshape(-1, 1),
        W_ih_r, (b_ih_r + b_hh_r).reshape(-1, 1), lin1_W,
        lin1_b.reshape(-1, 1), lin2_W, lin2_b.reshape(1, 1))

    x2 = jnp.stack([jnp.pad(fwd_x, ((0, NP - n), (0, CP - HID))),
                    jnp.pad(bwd_x, ((0, NP - n), (0, CP - HID)))])
    nid = _node_argmax(x2[:, :, :128])

    src = jnp.stack([fwd_edge_index[0], bwd_edge_index[0]])
    dst = jnp.stack([fwd_edge_index[1], bwd_edge_index[1]])

    padv = (10000 + (jnp.arange(NEP - NE, dtype=jnp.int32) % 240))
    src2d = jnp.concatenate(
        [jnp.concatenate([fwd_edge_index[0].astype(jnp.int32), padv]),
         jnp.concatenate([bwd_edge_index[0].astype(jnp.int32), padv])]
    ).reshape(NROW, 128)
    dst2d = jnp.concatenate(
        [jnp.concatenate([fwd_edge_index[1].astype(jnp.int32), padv]),
         jnp.concatenate([bwd_edge_index[1].astype(jnp.int32), padv])]
    ).reshape(NROW, 128)
    es2d = e_scalar.reshape(NROW, 128)

    x_upd_flat = _sc_scatter(x2.reshape(2 * NP * CP), src2d, dst2d, es2d,
                             nid.reshape(2 * NP))
    x_upd = x_upd_flat.reshape(2 * NP, CP)

    W_all = jnp.stack([jnp.stack([_pad2(fc_Wl), _pad2(fc_Wr)]),
                       jnp.stack([_pad2(bc_Wl), _pad2(bc_Wr)])])
    b_all = jnp.stack([jnp.stack([_pad1(fc_bl), _pad1(fc_br)]),
                       jnp.stack([_pad1(bc_bl), _pad1(bc_br)])])
    quarters = _projections(x_upd, W_all, b_all.reshape(2, 2, 1, CP))

    att2 = jnp.stack([_pad1(fc_att), _pad1(bc_att)])
    att_flat = jnp.concatenate(
        [att2[0, :192], att2[1, :192], att2[0, 192:], att2[1, 192:]])

    q0, q1, q2, q3 = quarters
    Lh = jnp.concatenate([jnp.concatenate([q0, q1], axis=1),
                          jnp.concatenate([q2, q3], axis=1)], axis=0)
    L4 = jnp.concatenate(quarters, axis=0)
    p = _sc_edge(Lh, src2d, dst2d, att_flat)
    acc, denom = _sc_accum(L4, src2d, dst2d, p)

    bias2 = jnp.stack([_pad1(fc_bias), _pad1(bc_bias)]).reshape(2, 1, CP)
    y = _final_combine(acc, denom, x_upd, bias2)
    return jnp.concatenate([y[:n, :HID], y[NP:NP + n, :HID]], axis=1)
